# Initial kernel scaffold; baseline (speedup 1.0000x reference)
import jax
import jax.numpy as jnp
from jax.experimental import pallas as pl


def kernel(anchor, negative):
    raise NotImplementedError("write your pallas kernel here")


# speedup vs baseline: 1.0906x; 1.0906x over previous
import functools

import jax
import jax.numpy as jnp
from jax import lax
from jax.experimental import pallas as pl
from jax.experimental.pallas import tpu as pltpu
from jax.experimental.pallas import tpu_sc as plsc

N = 4096
D = 32
BA = 256
NBLK = N // BA


def _bitrne(x):
    u = lax.bitcast_convert_type(x, jnp.uint32)
    ur = (u + jnp.uint32(0x7FFF) + ((u >> 16) & jnp.uint32(1))) & jnp.uint32(0xFFFF0000)
    return lax.bitcast_convert_type(ur, jnp.float32)


def _argmax_body(a_ref, n_ref, idx_ref):
    a = a_ref[...]
    n = n_ref[...]
    a2 = jnp.sum(a * a, axis=1)
    n2 = jnp.sum(n * n, axis=1)
    m = lax.dot_general(_bitrne(a), _bitrne(n), (((1,), (1,)), ((), ())),
                        preferred_element_type=jnp.float32)
    dist = (a2[:, None] - 2.0 * m + n2[None, :]) / D
    idx_ref[0, 0, :] = jnp.argmax(dist, axis=1).astype(jnp.int32)


_argmax_call = pl.pallas_call(
    _argmax_body,
    grid=(NBLK,),
    in_specs=[
        pl.BlockSpec((BA, D), lambda i: (i, 0)),
        pl.BlockSpec((N, D), lambda i: (0, 0)),
    ],
    out_specs=pl.BlockSpec((1, 1, BA), lambda i: (i, 0, 0)),
    out_shape=jax.ShapeDtypeStruct((NBLK, 1, BA), jnp.int32),
)


_mesh = plsc.VectorSubcoreMesh(core_axis_name="c", subcore_axis_name="s")
_NC, _NS = 2, 16
_NW = _NC * _NS
_BPW = N // _NW


@functools.partial(
    pl.kernel,
    mesh=_mesh,
    out_type=jax.ShapeDtypeStruct((N, D), jnp.float32),
    compiler_params=pltpu.CompilerParams(use_tc_tiling_on_sc=False),
    scratch_types=[
        pltpu.VMEM((_BPW,), jnp.int32),
        pltpu.VMEM((_BPW, D), jnp.float32),
        pltpu.SemaphoreType.DMA,
    ],
)
def _gather_call(neg_hbm, idx_hbm, out_hbm, idx_v, rows_v, sem):
    wid = lax.axis_index("s") * _NC + lax.axis_index("c")
    base = wid * _BPW
    pltpu.sync_copy(idx_hbm.at[pl.ds(base, _BPW)], idx_v)
    pltpu.async_copy(neg_hbm.at[idx_v], rows_v, sem).wait()
    pltpu.sync_copy(rows_v, out_hbm.at[pl.ds(base, _BPW)])


def kernel(anchor, negative):
    idx = _argmax_call(anchor, negative).reshape(N)
    return _gather_call(negative, idx)
